# Initial kernel scaffold; baseline (speedup 1.0000x reference)
import jax
import jax.numpy as jnp
from jax.experimental import pallas as pl


def kernel(k, beta, g, s, gamma, a, a_k, content_bias, key_bias, candidates):
    raise NotImplementedError("write your pallas kernel here")


# speedup vs baseline: 6.1039x; 6.1039x over previous
import jax
import jax.numpy as jnp
from jax.experimental import pallas as pl

_N = 8192
_B = 32
_K = 128
_M = 128
_TOPK = 5


def _gcl_kernel(kbT_ref, k_ref, beta_ref, gamma_ref, a_ref, content_ref, out_ref):
    kbT = kbT_ref[:, :]
    k = k_ref[:, :]
    beta = beta_ref[:, :]
    gamma = gamma_ref[:, :]
    a = a_ref[:, :]

    scores = jnp.dot(k, kbT, preferred_element_type=jnp.float32)
    rn = jnp.sqrt(jnp.sum(kbT * kbT, axis=0, keepdims=True))
    rk = jnp.sqrt(jnp.sum(k * k, axis=1, keepdims=True))
    denom = jnp.maximum(rn * rk, 1e-8)
    logits = beta * (scores / denom)

    m = jnp.max(logits, axis=1, keepdims=True)
    e = jnp.exp(logits - m)

    iota = jax.lax.broadcasted_iota(jnp.int32, (_B, _N), 1)
    cur = logits
    sel_mask = jnp.zeros((_B, _N), dtype=jnp.bool_)
    for _ in range(_TOPK):
        mv = jnp.max(cur, axis=1, keepdims=True)
        idx = jnp.min(jnp.where(cur == mv, iota, _N), axis=1, keepdims=True)
        sel = iota == idx
        sel_mask = jnp.logical_or(sel_mask, sel)
        cur = jnp.where(sel, -jnp.inf, cur)

    em = e * jnp.where(sel_mask, 1.0, 1e-16)
    wc = em / jnp.sum(em, axis=1, keepdims=True)
    w = jnp.exp(gamma * jnp.log(wc))
    w = w / jnp.sum(w, axis=1, keepdims=True)

    v = w - w * w
    sw2 = jnp.sum(w * w, axis=1, keepdims=True)
    out = jnp.dot(v, content_ref[:, :], preferred_element_type=jnp.float32)
    out_ref[:, :] = out + sw2 * a


def kernel(k, beta, g, s, gamma, a, a_k, content_bias, key_bias, candidates):
    del g, s, a_k, candidates
    kbT = key_bias.T
    return pl.pallas_call(
        _gcl_kernel,
        out_shape=jax.ShapeDtypeStruct((_B, _M), jnp.float32),
    )(kbT, k, beta, gamma, a, content_bias)


# speedup vs baseline: 8.6112x; 1.4108x over previous
import jax
import jax.numpy as jnp
from jax.experimental import pallas as pl

_N = 8192
_B = 32
_K = 128
_M = 128
_TOPK = 5

_NT = (((1,), (1,)), ((), ()))


def _gcl_kernel(kb_ref, k_ref, beta_ref, gamma_ref, a_ref, content_ref, out_ref):
    kb = kb_ref[:, :]
    k = k_ref[:, :]
    beta = beta_ref[:, :]
    gamma = gamma_ref[:, :]
    a = a_ref[:, :]

    scores = jax.lax.dot_general(k, kb, _NT, preferred_element_type=jnp.float32)
    ones = jnp.ones((1, _K), dtype=jnp.float32)
    rn2 = jax.lax.dot_general(ones, kb * kb, _NT, preferred_element_type=jnp.float32)
    rn = jnp.sqrt(rn2)
    rk = jnp.sqrt(jnp.sum(k * k, axis=1, keepdims=True))
    denom = jnp.maximum(rn * rk, 1e-8)
    logits = beta * (scores / denom)

    m = jnp.max(logits, axis=1, keepdims=True)
    t = logits - m
    e = jnp.exp(t)

    iota = jax.lax.broadcasted_iota(jnp.int32, (_B, _N), 1)
    cur = logits
    sel_mask = jnp.zeros((_B, _N), dtype=jnp.bool_)
    for _ in range(_TOPK):
        mv = jnp.max(cur, axis=1, keepdims=True)
        idx = jnp.min(jnp.where(cur == mv, iota, _N), axis=1, keepdims=True)
        sel = iota == idx
        sel_mask = jnp.logical_or(sel_mask, sel)
        cur = jnp.where(sel, -jnp.inf, cur)

    logf = jnp.where(sel_mask, 0.0, -36.8413614879047)
    em = e * jnp.where(sel_mask, 1.0, 1e-16)
    s1 = jnp.sum(em, axis=1, keepdims=True)
    w = jnp.exp(gamma * ((t + logf) - jnp.log(s1)))
    w = w / jnp.sum(w, axis=1, keepdims=True)

    v = w - w * w
    sw2 = jnp.sum(w * w, axis=1, keepdims=True)
    out = jnp.dot(v, content_ref[:, :], preferred_element_type=jnp.float32)
    out_ref[:, :] = out + sw2 * a


def kernel(k, beta, g, s, gamma, a, a_k, content_bias, key_bias, candidates):
    del g, s, a_k, candidates
    return pl.pallas_call(
        _gcl_kernel,
        out_shape=jax.ShapeDtypeStruct((_B, _M), jnp.float32),
    )(key_bias, k, beta, gamma, a, content_bias)
